# Initial kernel scaffold; baseline (speedup 1.0000x reference)
import jax
import jax.numpy as jnp
from jax.experimental import pallas as pl


def kernel(conv1_w, conv1_scale, conv1_bias, conv2, scale2, bias2, conv3, scale3, bias3, conv4, scale4, bias4, conv5, scale5, bias5, conv6, scale6, bias6, fc7, fc8, vote, x_nchw):
    raise NotImplementedError("write your pallas kernel here")


# speedup vs baseline: 1.5083x; 1.5083x over previous
import functools

import jax
import jax.numpy as jnp
from jax import lax
from jax.experimental import pallas as pl
from jax.experimental.pallas import tpu as pltpu

V_TH = 1.0


def _conv_stage(s, z_ref, a_ref, w_ref, sc_ref, bi_ref, H, W, C, pool):
    zero_col = jnp.zeros((H, 1, C), jnp.bfloat16)
    zero_row = jnp.zeros((1, W, 3 * C), jnp.bfloat16)
    left = jnp.concatenate([zero_col, s[:, : W - 1, :]], axis=1)
    right = jnp.concatenate([s[:, 1:, :], zero_col], axis=1)
    z = jnp.concatenate([left, s, right], axis=2)
    z_ref[...] = jnp.concatenate([zero_row, z, zero_row], axis=0)
    for dy in range(3):
        zs = z_ref[dy:dy + H, :, :].reshape(H * W, 3 * C)
        p = jnp.dot(zs, w_ref[dy], preferred_element_type=jnp.float32)
        if dy == 0:
            a_ref[...] = p
        else:
            a_ref[...] += p
    y = a_ref[...] * sc_ref[...] + bi_ref[...]
    spike = (y >= V_TH).astype(jnp.bfloat16).reshape(H, W, C)
    if pool:
        r = spike.reshape(H // 2, 2, W // 2, 2, C)
        r = jnp.max(r, axis=3)
        spike = jnp.max(r, axis=1)
    return spike


def _conv_net_kernel(x_ref,
                     w2, sc2, bi2, w3, sc3, bi3, w4, sc4, bi4,
                     w5, sc5, bi5, w6, sc6, bi6,
                     o_ref, z32, a32, z16, a16, *, C):
    s = x_ref[0]
    s = _conv_stage(s, z32, a32, w2, sc2, bi2, 32, 32, C, False)
    s = _conv_stage(s, z32, a32, w3, sc3, bi3, 32, 32, C, True)
    s = _conv_stage(s, z16, a16, w4, sc4, bi4, 16, 16, C, False)
    s = _conv_stage(s, z16, a16, w5, sc5, bi5, 16, 16, C, False)
    s = _conv_stage(s, z16, a16, w6, sc6, bi6, 16, 16, C, True)
    o_ref[0] = s


def _conv_net(x, convs, scales, biases):
    N, H, W, C = x.shape
    ws = [w.reshape(3, 3 * C, C) for w in convs]
    wspecs = []
    for _ in range(5):
        wspecs += [
            pl.BlockSpec((3, 3 * C, C), lambda g: (0, 0, 0)),
            pl.BlockSpec((1, C), lambda g: (0, 0)),
            pl.BlockSpec((1, C), lambda g: (0, 0)),
        ]
    args = []
    for w, sc, bi in zip(ws, scales, biases):
        args += [w, sc, bi]
    return pl.pallas_call(
        functools.partial(_conv_net_kernel, C=C),
        out_shape=jax.ShapeDtypeStruct((N, 8, 8, C), jnp.bfloat16),
        grid=(N,),
        in_specs=[pl.BlockSpec((1, H, W, C), lambda g: (g, 0, 0, 0))] + wspecs,
        out_specs=pl.BlockSpec((1, 8, 8, C), lambda g: (g, 0, 0, 0)),
        scratch_shapes=[
            pltpu.VMEM((H + 2, W, 3 * C), jnp.bfloat16),
            pltpu.VMEM((H * W, C), jnp.float32),
            pltpu.VMEM((H // 2 + 2, W // 2, 3 * C), jnp.bfloat16),
            pltpu.VMEM((H * W // 4, C), jnp.float32),
        ],
        compiler_params=pltpu.CompilerParams(
            dimension_semantics=("parallel",),
            vmem_limit_bytes=48 * 1024 * 1024),
    )(x, *args)


def _fc_if_kernel(x_ref, w_ref, o_ref, acc_ref):
    k = pl.program_id(1)

    @pl.when(k == 0)
    def _():
        acc_ref[...] = jnp.zeros_like(acc_ref)

    acc_ref[...] += jnp.dot(x_ref[...], w_ref[...], preferred_element_type=jnp.float32)

    @pl.when(k == pl.num_programs(1) - 1)
    def _():
        o_ref[...] = (acc_ref[...] >= V_TH).astype(o_ref.dtype)


def _fc_if(x, w_t):
    N, K = x.shape
    Cout = w_t.shape[1]
    tk = 4096 if K % 4096 == 0 else K
    tn = 1024 if Cout % 1024 == 0 else Cout
    return pl.pallas_call(
        _fc_if_kernel,
        out_shape=jax.ShapeDtypeStruct((N, Cout), jnp.bfloat16),
        grid=(Cout // tn, K // tk),
        in_specs=[
            pl.BlockSpec((N, tk), lambda j, k: (0, k)),
            pl.BlockSpec((tk, tn), lambda j, k: (k, j)),
        ],
        out_specs=pl.BlockSpec((N, tn), lambda j, k: (0, j)),
        scratch_shapes=[pltpu.VMEM((N, tn), jnp.float32)],
        compiler_params=pltpu.CompilerParams(
            dimension_semantics=("parallel", "arbitrary"),
            vmem_limit_bytes=40 * 1024 * 1024),
    )(x, w_t)


def _head_kernel(x_ref, w_ref, v_ref, o_ref):
    y = jnp.dot(x_ref[...], w_ref[...], preferred_element_type=jnp.float32)
    s = (y >= V_TH).astype(jnp.float32)
    o_ref[...] = jnp.dot(s, v_ref[...], preferred_element_type=jnp.float32)


def _head(x, w_t, vote):
    N, K = x.shape
    Cout = w_t.shape[1]
    Vout = vote.shape[1]
    return pl.pallas_call(
        _head_kernel,
        out_shape=jax.ShapeDtypeStruct((N, Vout), jnp.float32),
        grid=(1,),
        in_specs=[
            pl.BlockSpec((N, K), lambda j: (0, 0)),
            pl.BlockSpec((K, Cout), lambda j: (0, 0)),
            pl.BlockSpec((Cout, Vout), lambda j: (0, 0)),
        ],
        out_specs=pl.BlockSpec((N, Vout), lambda j: (0, 0)),
    )(x, w_t, vote)


def kernel(conv1_w, conv1_scale, conv1_bias, conv2, scale2, bias2,
           conv3, scale3, bias3, conv4, scale4, bias4, conv5, scale5, bias5,
           conv6, scale6, bias6, fc7, fc8, vote, x_nchw):
    x = jnp.transpose(x_nchw, (0, 2, 3, 1))
    y = lax.conv_general_dilated(x, conv1_w, (1, 1), "SAME",
                                 dimension_numbers=("NHWC", "HWIO", "NHWC"))
    y = y * conv1_scale + conv1_bias
    s1 = (y >= V_TH).astype(jnp.bfloat16)
    s = _conv_net(s1,
                  (conv2, conv3, conv4, conv5, conv6),
                  (scale2, scale3, scale4, scale5, scale6),
                  (bias2, bias3, bias4, bias5, bias6))
    n = s.shape[0]
    h = _fc_if(s.reshape(n, -1), fc7)
    return _head(h, fc8, vote)


# speedup vs baseline: 2.4108x; 1.5983x over previous
import functools

import jax
import jax.numpy as jnp
from jax import lax
from jax.experimental import pallas as pl
from jax.experimental.pallas import tpu as pltpu

V_TH = 1.0


def _conv_stage(s, z_ref, a_ref, w_ref, sc_ref, bi_ref, pool_ref, B, H, W, C):
    zero_col = jnp.zeros((B, H, 1, C), jnp.bfloat16)
    zero_row = jnp.zeros((B, 1, W, 3 * C), jnp.bfloat16)
    left = jnp.concatenate([zero_col, s[:, :, : W - 1, :]], axis=2)
    right = jnp.concatenate([s[:, :, 1:, :], zero_col], axis=2)
    z = jnp.concatenate([left, s, right], axis=3)
    z_ref[...] = jnp.concatenate([zero_row, z, zero_row], axis=1)
    for dy in range(3):
        zs = z_ref[:, dy:dy + H, :, :].reshape(B * H * W, 3 * C)
        p = jnp.dot(zs, w_ref[dy], preferred_element_type=jnp.float32)
        if dy == 0:
            a_ref[...] = p
        else:
            a_ref[...] += p
    y = a_ref[...] * sc_ref[...] + bi_ref[...]
    spike = (y >= V_TH).astype(jnp.bfloat16)
    if pool_ref is None:
        return spike.reshape(B, H, W, C)
    sp = spike.reshape(B, H * W, C)
    pm = pool_ref[...]
    outs = []
    for b in range(B):
        cnt = jnp.dot(pm, sp[b], preferred_element_type=jnp.float32)
        outs.append((cnt >= 0.5).astype(jnp.bfloat16))
    return jnp.stack(outs, axis=0).reshape(B, H // 2, W // 2, C)


def _conv_net_kernel(x_ref,
                     w2, sc2, bi2, w3, sc3, bi3, w4, sc4, bi4,
                     w5, sc5, bi5, w6, sc6, bi6, p32, p16,
                     o_ref, z32a, a32a, z16a, a16a, z32b, a32b, z16b, a16b,
                     *, B, C):
    hb = B // 2
    sa = x_ref[0:hb]
    sb = x_ref[hb:B]
    stages = [(w2, sc2, bi2, None, 32), (w3, sc3, bi3, p32, 32),
              (w4, sc4, bi4, None, 16), (w5, sc5, bi5, None, 16),
              (w6, sc6, bi6, p16, 16)]
    for w, sc, bi, pm, hw in stages:
        za, aa = (z32a, a32a) if hw == 32 else (z16a, a16a)
        zb, ab = (z32b, a32b) if hw == 32 else (z16b, a16b)
        sa = _conv_stage(sa, za, aa, w, sc, bi, pm, hb, hw, hw, C)
        sb = _conv_stage(sb, zb, ab, w, sc, bi, pm, hb, hw, hw, C)
    o_ref[0:hb] = sa
    o_ref[hb:B] = sb


def _pool_matrix(H, W):
    ho = jnp.arange(H // 2 * W // 2) // (W // 2)
    wo = jnp.arange(H // 2 * W // 2) % (W // 2)
    h = jnp.arange(H * W) // W
    w = jnp.arange(H * W) % W
    sel = ((h[None, :] // 2 == ho[:, None]) & (w[None, :] // 2 == wo[:, None]))
    return sel.astype(jnp.bfloat16)


def _conv_net(x, convs, scales, biases, B=4):
    N, H, W, C = x.shape
    if N % B != 0:
        B = 2
    assert N % B == 0
    ws = [w.reshape(3, 3 * C, C) for w in convs]
    wspecs = []
    for _ in range(5):
        wspecs += [
            pl.BlockSpec((3, 3 * C, C), lambda g: (0, 0, 0)),
            pl.BlockSpec((1, C), lambda g: (0, 0)),
            pl.BlockSpec((1, C), lambda g: (0, 0)),
        ]
    args = []
    for w, sc, bi in zip(ws, scales, biases):
        args += [w, sc, bi]
    p32 = _pool_matrix(H, W)
    p16 = _pool_matrix(H // 2, W // 2)
    pspecs = [
        pl.BlockSpec(p32.shape, lambda g: (0, 0)),
        pl.BlockSpec(p16.shape, lambda g: (0, 0)),
    ]
    return pl.pallas_call(
        functools.partial(_conv_net_kernel, B=B, C=C),
        out_shape=jax.ShapeDtypeStruct((N, 8, 8, C), jnp.bfloat16),
        grid=(N // B,),
        in_specs=[pl.BlockSpec((B, H, W, C), lambda g: (g, 0, 0, 0))]
                 + wspecs + pspecs,
        out_specs=pl.BlockSpec((B, 8, 8, C), lambda g: (g, 0, 0, 0)),
        scratch_shapes=[
            pltpu.VMEM((B // 2, H + 2, W, 3 * C), jnp.bfloat16),
            pltpu.VMEM((B // 2 * H * W, C), jnp.float32),
            pltpu.VMEM((B // 2, H // 2 + 2, W // 2, 3 * C), jnp.bfloat16),
            pltpu.VMEM((B // 2 * H * W // 4, C), jnp.float32),
            pltpu.VMEM((B // 2, H + 2, W, 3 * C), jnp.bfloat16),
            pltpu.VMEM((B // 2 * H * W, C), jnp.float32),
            pltpu.VMEM((B // 2, H // 2 + 2, W // 2, 3 * C), jnp.bfloat16),
            pltpu.VMEM((B // 2 * H * W // 4, C), jnp.float32),
        ],
        compiler_params=pltpu.CompilerParams(
            dimension_semantics=("parallel",),
            vmem_limit_bytes=56 * 1024 * 1024),
    )(x, *args, p32, p16)


def _fc_if_kernel(x_ref, w_ref, o_ref, acc_ref):
    k = pl.program_id(1)

    @pl.when(k == 0)
    def _():
        acc_ref[...] = jnp.zeros_like(acc_ref)

    acc_ref[...] += jnp.dot(x_ref[...], w_ref[...], preferred_element_type=jnp.float32)

    @pl.when(k == pl.num_programs(1) - 1)
    def _():
        o_ref[...] = (acc_ref[...] >= V_TH).astype(o_ref.dtype)


def _fc_if(x, w_t):
    N, K = x.shape
    Cout = w_t.shape[1]
    tk = 4096 if K % 4096 == 0 else K
    tn = 1024 if Cout % 1024 == 0 else Cout
    return pl.pallas_call(
        _fc_if_kernel,
        out_shape=jax.ShapeDtypeStruct((N, Cout), jnp.bfloat16),
        grid=(Cout // tn, K // tk),
        in_specs=[
            pl.BlockSpec((N, tk), lambda j, k: (0, k)),
            pl.BlockSpec((tk, tn), lambda j, k: (k, j)),
        ],
        out_specs=pl.BlockSpec((N, tn), lambda j, k: (0, j)),
        scratch_shapes=[pltpu.VMEM((N, tn), jnp.float32)],
        compiler_params=pltpu.CompilerParams(
            dimension_semantics=("parallel", "arbitrary"),
            vmem_limit_bytes=40 * 1024 * 1024),
    )(x, w_t)


def _head_kernel(x_ref, w_ref, v_ref, o_ref):
    y = jnp.dot(x_ref[...], w_ref[...], preferred_element_type=jnp.float32)
    s = (y >= V_TH).astype(jnp.float32)
    o_ref[...] = jnp.dot(s, v_ref[...], preferred_element_type=jnp.float32)


def _head(x, w_t, vote):
    N, K = x.shape
    Cout = w_t.shape[1]
    Vout = vote.shape[1]
    return pl.pallas_call(
        _head_kernel,
        out_shape=jax.ShapeDtypeStruct((N, Vout), jnp.float32),
        grid=(1,),
        in_specs=[
            pl.BlockSpec((N, K), lambda j: (0, 0)),
            pl.BlockSpec((K, Cout), lambda j: (0, 0)),
            pl.BlockSpec((Cout, Vout), lambda j: (0, 0)),
        ],
        out_specs=pl.BlockSpec((N, Vout), lambda j: (0, 0)),
    )(x, w_t, vote)


def kernel(conv1_w, conv1_scale, conv1_bias, conv2, scale2, bias2,
           conv3, scale3, bias3, conv4, scale4, bias4, conv5, scale5, bias5,
           conv6, scale6, bias6, fc7, fc8, vote, x_nchw):
    x = jnp.transpose(x_nchw, (0, 2, 3, 1))
    y = lax.conv_general_dilated(x, conv1_w, (1, 1), "SAME",
                                 dimension_numbers=("NHWC", "HWIO", "NHWC"))
    y = y * conv1_scale + conv1_bias
    s1 = (y >= V_TH).astype(jnp.bfloat16)
    s = _conv_net(s1,
                  (conv2, conv3, conv4, conv5, conv6),
                  (scale2, scale3, scale4, scale5, scale6),
                  (bias2, bias3, bias4, bias5, bias6))
    n = s.shape[0]
    h = _fc_if(s.reshape(n, -1), fc7)
    return _head(h, fc8, vote)
